# Initial kernel scaffold; baseline (speedup 1.0000x reference)
import jax
import jax.numpy as jnp
from jax.experimental import pallas as pl


def kernel(x, edge_index, d, idx, W1, gamma1, beta1, W2, gamma2, beta2, fc_W, fc_b):
    raise NotImplementedError("write your pallas kernel here")


# speedup vs baseline: 3.7562x; 3.7562x over previous
import functools

import jax
import jax.numpy as jnp
from jax import lax
from jax.experimental import pallas as pl
from jax.experimental.pallas import tpu as pltpu
from jax.experimental.pallas import tpu_sc as plsc

N = 10000
E = 320000
DIM = 128
NCLASS = 7
MAX_DEGREE = 64
SAM = 1024

NC = 2
NS = 16
NW = NC * NS
EPT = E // NW
CB = 50
KC = EPT // CB
RPT = N // NS

BR = 1000
GRID = N // BR


def _sc_body(do_hist, h_hbm, sd_hbm, idx_hbm, zeros_hbm, ones_hbm, e1_hbm,
             *rest):
    if do_hist:
        (out_hbm, cnt_hbm, acc_sh, i0b, i1b, buf0, buf1,
         ones_v, e1_v, idx_v, g0, g1, i0s, i1s) = rest
    else:
        out_hbm, acc_sh, i0b, i1b, buf0, buf1, g0, g1, i0s, i1s = rest
    c = lax.axis_index("c")
    s = lax.axis_index("s")
    wid = c * NS + s

    pltpu.sync_copy(zeros_hbm, acc_sh.at[pl.ds(s * RPT, RPT)])
    if do_hist:
        pltpu.sync_copy(ones_hbm, ones_v)
        pltpu.sync_copy(e1_hbm, e1_v)
        pltpu.sync_copy(idx_hbm.at[s], idx_v)
    plsc.subcore_barrier()

    def step(j, _):
        pltpu.sync_copy(sd_hbm.at[wid, j], i0b)
        pltpu.async_copy(h_hbm.at[i0b.at[0]], buf0, g0).wait()
        pltpu.sync_copy(buf0, acc_sh.at[i0b.at[1]], add=True)
        return 0

    lax.fori_loop(0, KC, step, 0)
    plsc.subcore_barrier()
    pltpu.sync_copy(acc_sh.at[pl.ds(s * RPT, RPT)], out_hbm.at[c, s])
    plsc.subcore_barrier()

    if do_hist:
        pltpu.sync_copy(zeros_hbm, acc_sh.at[pl.ds(s * RPT, RPT)])
        plsc.subcore_barrier()

        def step2(j, _):
            pltpu.sync_copy(sd_hbm.at[wid, j], i0b)
            pltpu.sync_copy(ones_v, acc_sh.at[i0b.at[1]], add=True)
            return 0

        lax.fori_loop(0, KC, step2, 0)

        @pl.when(c == 0)
        def _():
            pltpu.sync_copy(e1_v, acc_sh.at[idx_v.at[0]], add=True)

        plsc.subcore_barrier()
        pltpu.sync_copy(acc_sh.at[pl.ds(s * RPT, RPT)], cnt_hbm.at[c, s])


def _make_sc_kernel(do_hist):
    mesh = plsc.VectorSubcoreMesh(core_axis_name="c", subcore_axis_name="s",
                                  num_cores=NC, num_subcores=NS)
    out_type = [
        jax.ShapeDtypeStruct((NC, NS, RPT, DIM), jnp.float32),
    ]
    scratch = [
        pltpu.VMEM_SHARED((N, DIM), jnp.float32),
        pltpu.VMEM((2, CB), jnp.int32),
        pltpu.VMEM((2, CB), jnp.int32),
        pltpu.VMEM((CB, DIM), jnp.float32),
        pltpu.VMEM((CB, DIM), jnp.float32),
    ]
    if do_hist:
        out_type.append(
            jax.ShapeDtypeStruct((NC, NS, RPT, DIM), jnp.float32))
        scratch += [
            pltpu.VMEM((CB, DIM), jnp.float32),
            pltpu.VMEM((64, DIM), jnp.float32),
            pltpu.VMEM((1, 64), jnp.int32),
        ]
    scratch += [pltpu.SemaphoreType.DMA] * 4
    return pl.kernel(functools.partial(_sc_body, do_hist), out_type=out_type,
                     mesh=mesh, scratch_types=scratch)


def _mm_body(x_ref, w_ref, o_ref):
    o_ref[...] = jnp.dot(x_ref[...], w_ref[...],
                         preferred_element_type=jnp.float32)


def _input_matmul(x, W):
    return pl.pallas_call(
        _mm_body,
        grid=(GRID,),
        in_specs=[pl.BlockSpec((BR, DIM), lambda i: (i, 0)),
                  pl.BlockSpec((DIM, DIM), lambda i: (0, 0))],
        out_specs=pl.BlockSpec((BR, DIM), lambda i: (i, 0)),
        out_shape=jax.ShapeDtypeStruct((N, DIM), jnp.float32),
    )(x, W)


def _film_body(p_ref, cnt_ref, hpre_ref, d_ref, g_ref, b_ref,
               w2_ref, bias_ref, h_ref, hnext_ref, lb_ref, lf_ref,
               u_acc, q_acc):
    i = pl.program_id(0)

    @pl.when(i == 0)
    def _():
        u_acc[...] = jnp.zeros_like(u_acc)
        q_acc[...] = jnp.zeros_like(q_acc)

    cnt = cnt_ref[0, :, 0:1] + cnt_ref[1, :, 0:1]
    sw = cnt_ref[0, :, 1:2]
    agg = (p_ref[0] + p_ref[1]) / jnp.maximum(cnt, 1.0)
    iota = lax.broadcasted_iota(jnp.int32, (BR, MAX_DEGREE), 1)
    onehot = (d_ref[...] == iota.astype(jnp.float32)).astype(jnp.float32)
    gamma = jnp.dot(onehot, g_ref[...], preferred_element_type=jnp.float32)
    beta = jnp.dot(onehot, b_ref[...], preferred_element_type=jnp.float32)
    out = gamma * agg + beta + hpre_ref[...]

    u_acc[...] += jnp.sum(sw * out, axis=0, keepdims=True)
    q_acc[...] += jnp.sum(sw * out * out, axis=0, keepdims=True)

    h = jnp.where(out > 0, out, 0.01 * out)
    h_ref[...] = h
    hnext_ref[...] = (jnp.dot(h, w2_ref[...],
                              preferred_element_type=jnp.float32)
                      + bias_ref[...])

    @pl.when(i == GRID - 1)
    def _():
        u = u_acc[...]
        q = q_acc[...]
        lb = (jnp.sum(q) - jnp.sum(u * u) / SAM) / (SAM * DIM)
        lf = ((jnp.sum(g_ref[...] * g_ref[...])
               + jnp.sum(b_ref[...] * b_ref[...]))
              / (MAX_DEGREE * DIM))
        lb_ref[...] = lb.reshape(1, 1)
        lf_ref[...] = lf.reshape(1, 1)


def _film_layer(parts, cnts, h_pre, d_f, gemb, bemb, W_next, bias_next):
    return pl.pallas_call(
        _film_body,
        grid=(GRID,),
        in_specs=[
            pl.BlockSpec((NC, BR, DIM), lambda i: (0, i, 0)),
            pl.BlockSpec((NC, BR, DIM), lambda i: (0, i, 0)),
            pl.BlockSpec((BR, DIM), lambda i: (i, 0)),
            pl.BlockSpec((BR, 1), lambda i: (i, 0)),
            pl.BlockSpec((MAX_DEGREE, DIM), lambda i: (0, 0)),
            pl.BlockSpec((MAX_DEGREE, DIM), lambda i: (0, 0)),
            pl.BlockSpec((DIM, DIM), lambda i: (0, 0)),
            pl.BlockSpec((1, DIM), lambda i: (0, 0)),
        ],
        out_specs=[
            pl.BlockSpec((BR, DIM), lambda i: (i, 0)),
            pl.BlockSpec((BR, DIM), lambda i: (i, 0)),
            pl.BlockSpec((1, 1), lambda i: (0, 0)),
            pl.BlockSpec((1, 1), lambda i: (0, 0)),
        ],
        out_shape=[
            jax.ShapeDtypeStruct((N, DIM), jnp.float32),
            jax.ShapeDtypeStruct((N, DIM), jnp.float32),
            jax.ShapeDtypeStruct((1, 1), jnp.float32),
            jax.ShapeDtypeStruct((1, 1), jnp.float32),
        ],
        scratch_shapes=[
            pltpu.VMEM((1, DIM), jnp.float32),
            pltpu.VMEM((1, DIM), jnp.float32),
        ],
    )(parts, cnts, h_pre, d_f, gemb, bemb, W_next, bias_next)


def kernel(x, edge_index, d, idx, W1, gamma1, beta1, W2, gamma2, beta2,
           fc_W, fc_b):
    src3 = edge_index[0].reshape(NW, KC, CB)
    dst3 = edge_index[1].reshape(NW, KC, CB)
    sd3 = jnp.stack((src3, dst3), axis=2)
    idx2 = idx.reshape(NS, 1, 64)
    d_f = d.astype(jnp.float32).reshape(N, 1)
    zeros = jnp.zeros((RPT, DIM), jnp.float32)
    ones0 = jnp.zeros((CB, DIM), jnp.float32).at[:, 0].set(1.0)
    e1 = jnp.zeros((64, DIM), jnp.float32).at[:, 1].set(1.0)
    fcWp = jnp.zeros((DIM, DIM), jnp.float32).at[:, :NCLASS].set(fc_W)
    fcbp = jnp.zeros((1, DIM), jnp.float32).at[0, :NCLASS].set(fc_b)
    zbias = jnp.zeros((1, DIM), jnp.float32)

    sc1 = _make_sc_kernel(True)
    sc2 = _make_sc_kernel(False)

    h1_pre = _input_matmul(x, W1)
    parts1, cnts = sc1(h1_pre, sd3, idx2, zeros, ones0, e1)
    parts1 = parts1.reshape(NC, N, DIM)
    cnts = cnts.reshape(NC, N, DIM)
    h1, h2_pre, b1, f1 = _film_layer(parts1, cnts, h1_pre, d_f,
                                     gamma1, beta1, W2, zbias)
    parts2 = sc2(h2_pre, sd3, idx2, zeros, ones0, e1)
    if isinstance(parts2, (list, tuple)):
        parts2 = parts2[0]
    parts2 = parts2.reshape(NC, N, DIM)
    _, logits_pad, b2, f2 = _film_layer(parts2, cnts, h2_pre, d_f,
                                        gamma2, beta2, fcWp, fcbp)

    logits = logits_pad[:, :NCLASS]
    return (logits, (b1 + b2)[0, 0], (f1 + f2)[0, 0], h1)


# speedup vs baseline: 5.8937x; 1.5691x over previous
import functools

import jax
import jax.numpy as jnp
from jax import lax
from jax.experimental import pallas as pl
from jax.experimental.pallas import tpu as pltpu
from jax.experimental.pallas import tpu_sc as plsc

N = 10000
E = 320000
DIM = 128
NCLASS = 7
MAX_DEGREE = 64
SAM = 1024

NC = 2
NS = 16
NW = NC * NS
EPT = E // NW
CB = 50
KC = EPT // CB
RPT = N // NS

BR = 1000
GRID = N // BR


def _sc_body(do_hist, h_hbm, sd_hbm, idx_hbm, zeros_hbm, ones_hbm, e1_hbm,
             *rest):
    if do_hist:
        (out_hbm, cnt_hbm, acc_sh, i0b, i1b, buf0, buf1,
         ones_v, e1_v, idx_v, g0, g1, i0s, i1s) = rest
    else:
        out_hbm, acc_sh, i0b, i1b, buf0, buf1, g0, g1, i0s, i1s = rest
    c = lax.axis_index("c")
    s = lax.axis_index("s")
    wid = c * NS + s

    pltpu.sync_copy(zeros_hbm, acc_sh.at[pl.ds(s * RPT, RPT)])
    if do_hist:
        pltpu.sync_copy(ones_hbm, ones_v)
        pltpu.sync_copy(e1_hbm, e1_v)
        pltpu.sync_copy(idx_hbm.at[s], idx_v)
    plsc.subcore_barrier()

    pltpu.sync_copy(sd_hbm.at[wid, 0], i0b)
    pltpu.sync_copy(sd_hbm.at[wid, 1], i1b)
    pltpu.async_copy(h_hbm.at[i0b.at[0]], buf0, g0)
    last = KC // 2 - 1

    def step(i, _):
        j0 = 2 * i

        @pl.when(i > 0)
        def _():
            pltpu.make_async_copy(sd_hbm.at[wid, 0], i1b, i1s).wait()

        pltpu.async_copy(h_hbm.at[i1b.at[0]], buf1, g1)
        pltpu.make_async_copy(h_hbm.at[i0b.at[0]], buf0, g0).wait()
        pltpu.sync_copy(buf0, acc_sh.at[i0b.at[1]], add=True)

        @pl.when(i < last)
        def _():
            pltpu.async_copy(sd_hbm.at[wid, j0 + 2], i0b, i0s)

        pltpu.make_async_copy(h_hbm.at[i1b.at[0]], buf1, g1).wait()
        pltpu.sync_copy(buf1, acc_sh.at[i1b.at[1]], add=True)

        @pl.when(i < last)
        def _():
            pltpu.async_copy(sd_hbm.at[wid, j0 + 3], i1b, i1s)
            pltpu.make_async_copy(sd_hbm.at[wid, 0], i0b, i0s).wait()
            pltpu.async_copy(h_hbm.at[i0b.at[0]], buf0, g0)

        return 0

    lax.fori_loop(0, KC // 2, step, 0)
    plsc.subcore_barrier()
    pltpu.sync_copy(acc_sh.at[pl.ds(s * RPT, RPT)], out_hbm.at[c, s])
    plsc.subcore_barrier()

    if do_hist:
        pltpu.sync_copy(zeros_hbm, acc_sh.at[pl.ds(s * RPT, RPT)])
        plsc.subcore_barrier()

        pltpu.sync_copy(sd_hbm.at[wid, 0], i0b)
        pltpu.sync_copy(sd_hbm.at[wid, 1], i1b)

        def step2(i, _):
            j0 = 2 * i

            @pl.when(i > 0)
            def _():
                pltpu.make_async_copy(sd_hbm.at[wid, 0], i1b, i1s).wait()

            pltpu.sync_copy(ones_v, acc_sh.at[i0b.at[1]], add=True)

            @pl.when(i < last)
            def _():
                pltpu.async_copy(sd_hbm.at[wid, j0 + 2], i0b, i0s)

            pltpu.sync_copy(ones_v, acc_sh.at[i1b.at[1]], add=True)

            @pl.when(i < last)
            def _():
                pltpu.async_copy(sd_hbm.at[wid, j0 + 3], i1b, i1s)
                pltpu.make_async_copy(sd_hbm.at[wid, 0], i0b, i0s).wait()

            return 0

        lax.fori_loop(0, KC // 2, step2, 0)

        @pl.when(c == 0)
        def _():
            pltpu.sync_copy(e1_v, acc_sh.at[idx_v.at[0]], add=True)

        plsc.subcore_barrier()
        pltpu.sync_copy(acc_sh.at[pl.ds(s * RPT, RPT)], cnt_hbm.at[c, s])


def _make_sc_kernel(do_hist):
    mesh = plsc.VectorSubcoreMesh(core_axis_name="c", subcore_axis_name="s",
                                  num_cores=NC, num_subcores=NS)
    out_type = [
        jax.ShapeDtypeStruct((NC, NS, RPT, DIM), jnp.float32),
    ]
    scratch = [
        pltpu.VMEM_SHARED((N, DIM), jnp.float32),
        pltpu.VMEM((2, CB), jnp.int32),
        pltpu.VMEM((2, CB), jnp.int32),
        pltpu.VMEM((CB, DIM), jnp.float32),
        pltpu.VMEM((CB, DIM), jnp.float32),
    ]
    if do_hist:
        out_type.append(
            jax.ShapeDtypeStruct((NC, NS, RPT, DIM), jnp.float32))
        scratch += [
            pltpu.VMEM((CB, DIM), jnp.float32),
            pltpu.VMEM((64, DIM), jnp.float32),
            pltpu.VMEM((1, 64), jnp.int32),
        ]
    scratch += [pltpu.SemaphoreType.DMA] * 4
    return pl.kernel(functools.partial(_sc_body, do_hist), out_type=out_type,
                     mesh=mesh, scratch_types=scratch)


def _mm_body(x_ref, w_ref, o_ref):
    o_ref[...] = jnp.dot(x_ref[...], w_ref[...],
                         preferred_element_type=jnp.float32)


def _input_matmul(x, W):
    return pl.pallas_call(
        _mm_body,
        grid=(GRID,),
        in_specs=[pl.BlockSpec((BR, DIM), lambda i: (i, 0)),
                  pl.BlockSpec((DIM, DIM), lambda i: (0, 0))],
        out_specs=pl.BlockSpec((BR, DIM), lambda i: (i, 0)),
        out_shape=jax.ShapeDtypeStruct((N, DIM), jnp.float32),
    )(x, W)


def _film_body(p_ref, cnt_ref, hpre_ref, d_ref, g_ref, b_ref,
               w2_ref, bias_ref, h_ref, hnext_ref, lb_ref, lf_ref,
               u_acc, q_acc):
    i = pl.program_id(0)

    @pl.when(i == 0)
    def _():
        u_acc[...] = jnp.zeros_like(u_acc)
        q_acc[...] = jnp.zeros_like(q_acc)

    cnt = cnt_ref[0, :, 0:1] + cnt_ref[1, :, 0:1]
    sw = cnt_ref[0, :, 1:2]
    agg = (p_ref[0] + p_ref[1]) / jnp.maximum(cnt, 1.0)
    iota = lax.broadcasted_iota(jnp.int32, (BR, MAX_DEGREE), 1)
    onehot = (d_ref[...] == iota.astype(jnp.float32)).astype(jnp.float32)
    gamma = jnp.dot(onehot, g_ref[...], preferred_element_type=jnp.float32)
    beta = jnp.dot(onehot, b_ref[...], preferred_element_type=jnp.float32)
    out = gamma * agg + beta + hpre_ref[...]

    u_acc[...] += jnp.sum(sw * out, axis=0, keepdims=True)
    q_acc[...] += jnp.sum(sw * out * out, axis=0, keepdims=True)

    h = jnp.where(out > 0, out, 0.01 * out)
    h_ref[...] = h
    hnext_ref[...] = (jnp.dot(h, w2_ref[...],
                              preferred_element_type=jnp.float32)
                      + bias_ref[...])

    @pl.when(i == GRID - 1)
    def _():
        u = u_acc[...]
        q = q_acc[...]
        lb = (jnp.sum(q) - jnp.sum(u * u) / SAM) / (SAM * DIM)
        lf = ((jnp.sum(g_ref[...] * g_ref[...])
               + jnp.sum(b_ref[...] * b_ref[...]))
              / (MAX_DEGREE * DIM))
        lb_ref[...] = lb.reshape(1, 1)
        lf_ref[...] = lf.reshape(1, 1)


def _film_layer(parts, cnts, h_pre, d_f, gemb, bemb, W_next, bias_next):
    return pl.pallas_call(
        _film_body,
        grid=(GRID,),
        in_specs=[
            pl.BlockSpec((NC, BR, DIM), lambda i: (0, i, 0)),
            pl.BlockSpec((NC, BR, DIM), lambda i: (0, i, 0)),
            pl.BlockSpec((BR, DIM), lambda i: (i, 0)),
            pl.BlockSpec((BR, 1), lambda i: (i, 0)),
            pl.BlockSpec((MAX_DEGREE, DIM), lambda i: (0, 0)),
            pl.BlockSpec((MAX_DEGREE, DIM), lambda i: (0, 0)),
            pl.BlockSpec((DIM, DIM), lambda i: (0, 0)),
            pl.BlockSpec((1, DIM), lambda i: (0, 0)),
        ],
        out_specs=[
            pl.BlockSpec((BR, DIM), lambda i: (i, 0)),
            pl.BlockSpec((BR, DIM), lambda i: (i, 0)),
            pl.BlockSpec((1, 1), lambda i: (0, 0)),
            pl.BlockSpec((1, 1), lambda i: (0, 0)),
        ],
        out_shape=[
            jax.ShapeDtypeStruct((N, DIM), jnp.float32),
            jax.ShapeDtypeStruct((N, DIM), jnp.float32),
            jax.ShapeDtypeStruct((1, 1), jnp.float32),
            jax.ShapeDtypeStruct((1, 1), jnp.float32),
        ],
        scratch_shapes=[
            pltpu.VMEM((1, DIM), jnp.float32),
            pltpu.VMEM((1, DIM), jnp.float32),
        ],
    )(parts, cnts, h_pre, d_f, gemb, bemb, W_next, bias_next)


def kernel(x, edge_index, d, idx, W1, gamma1, beta1, W2, gamma2, beta2,
           fc_W, fc_b):
    src3 = edge_index[0].reshape(NW, KC, CB)
    dst3 = edge_index[1].reshape(NW, KC, CB)
    sd3 = jnp.stack((src3, dst3), axis=2)
    idx2 = idx.reshape(NS, 1, 64)
    d_f = d.astype(jnp.float32).reshape(N, 1)
    zeros = jnp.zeros((RPT, DIM), jnp.float32)
    ones0 = jnp.zeros((CB, DIM), jnp.float32).at[:, 0].set(1.0)
    e1 = jnp.zeros((64, DIM), jnp.float32).at[:, 1].set(1.0)
    fcWp = jnp.zeros((DIM, DIM), jnp.float32).at[:, :NCLASS].set(fc_W)
    fcbp = jnp.zeros((1, DIM), jnp.float32).at[0, :NCLASS].set(fc_b)
    zbias = jnp.zeros((1, DIM), jnp.float32)

    sc1 = _make_sc_kernel(True)
    sc2 = _make_sc_kernel(False)

    h1_pre = _input_matmul(x, W1)
    parts1, cnts = sc1(h1_pre, sd3, idx2, zeros, ones0, e1)
    parts1 = parts1.reshape(NC, N, DIM)
    cnts = cnts.reshape(NC, N, DIM)
    h1, h2_pre, b1, f1 = _film_layer(parts1, cnts, h1_pre, d_f,
                                     gamma1, beta1, W2, zbias)
    parts2 = sc2(h2_pre, sd3, idx2, zeros, ones0, e1)
    if isinstance(parts2, (list, tuple)):
        parts2 = parts2[0]
    parts2 = parts2.reshape(NC, N, DIM)
    _, logits_pad, b2, f2 = _film_layer(parts2, cnts, h2_pre, d_f,
                                        gamma2, beta2, fcWp, fcbp)

    logits = logits_pad[:, :NCLASS]
    return (logits, (b1 + b2)[0, 0], (f1 + f2)[0, 0], h1)


# speedup vs baseline: 7.6126x; 1.2916x over previous
import functools

import jax
import jax.numpy as jnp
from jax import lax
from jax.experimental import pallas as pl
from jax.experimental.pallas import tpu as pltpu
from jax.experimental.pallas import tpu_sc as plsc

N = 10000
E = 320000
DIM = 128
NCLASS = 7
MAX_DEGREE = 64
SAM = 1024

NC = 2
NS = 16
NW = NC * NS
EPT = E // NW
CB = 100
KC = EPT // CB
RPT = N // NS

BR = 1000
GRID = N // BR


def _sc_body(do_hist, h_hbm, sd_hbm, idx_hbm, zeros_hbm, ones_hbm, e1_hbm,
             *rest):
    if do_hist:
        (out_hbm, cnt_hbm, acc_sh, i0b, i1b, buf0, buf1,
         ones_v, e1_v, idx_v, g0, g1, i0s, i1s) = rest
    else:
        out_hbm, acc_sh, i0b, i1b, buf0, buf1, g0, g1, i0s, i1s = rest
    c = lax.axis_index("c")
    s = lax.axis_index("s")
    wid = c * NS + s

    pltpu.sync_copy(zeros_hbm, acc_sh.at[pl.ds(s * RPT, RPT)])
    if do_hist:
        pltpu.sync_copy(ones_hbm, ones_v)
        pltpu.sync_copy(e1_hbm, e1_v)
        pltpu.sync_copy(idx_hbm.at[s], idx_v)
    plsc.subcore_barrier()

    pltpu.sync_copy(sd_hbm.at[wid, 0], i0b)
    pltpu.sync_copy(sd_hbm.at[wid, 1], i1b)
    pltpu.async_copy(h_hbm.at[i0b.at[0]], buf0, g0)
    last = KC // 2 - 1

    def step(i, _):
        j0 = 2 * i

        @pl.when(i > 0)
        def _():
            pltpu.make_async_copy(sd_hbm.at[wid, 0], i1b, i1s).wait()

        pltpu.async_copy(h_hbm.at[i1b.at[0]], buf1, g1)
        pltpu.make_async_copy(h_hbm.at[i0b.at[0]], buf0, g0).wait()
        pltpu.sync_copy(buf0, acc_sh.at[i0b.at[1]], add=True)

        @pl.when(i < last)
        def _():
            pltpu.async_copy(sd_hbm.at[wid, j0 + 2], i0b, i0s)

        pltpu.make_async_copy(h_hbm.at[i1b.at[0]], buf1, g1).wait()
        pltpu.sync_copy(buf1, acc_sh.at[i1b.at[1]], add=True)

        @pl.when(i < last)
        def _():
            pltpu.async_copy(sd_hbm.at[wid, j0 + 3], i1b, i1s)
            pltpu.make_async_copy(sd_hbm.at[wid, 0], i0b, i0s).wait()
            pltpu.async_copy(h_hbm.at[i0b.at[0]], buf0, g0)

        return 0

    lax.fori_loop(0, KC // 2, step, 0)
    plsc.subcore_barrier()
    pltpu.sync_copy(acc_sh.at[pl.ds(s * RPT, RPT)], out_hbm.at[c, s])
    plsc.subcore_barrier()

    if do_hist:
        pltpu.sync_copy(zeros_hbm, acc_sh.at[pl.ds(s * RPT, RPT)])
        plsc.subcore_barrier()

        pltpu.sync_copy(sd_hbm.at[wid, 0], i0b)
        pltpu.sync_copy(sd_hbm.at[wid, 1], i1b)

        def step2(i, _):
            j0 = 2 * i

            @pl.when(i > 0)
            def _():
                pltpu.make_async_copy(sd_hbm.at[wid, 0], i1b, i1s).wait()

            pltpu.sync_copy(ones_v, acc_sh.at[i0b.at[1]], add=True)

            @pl.when(i < last)
            def _():
                pltpu.async_copy(sd_hbm.at[wid, j0 + 2], i0b, i0s)

            pltpu.sync_copy(ones_v, acc_sh.at[i1b.at[1]], add=True)

            @pl.when(i < last)
            def _():
                pltpu.async_copy(sd_hbm.at[wid, j0 + 3], i1b, i1s)
                pltpu.make_async_copy(sd_hbm.at[wid, 0], i0b, i0s).wait()

            return 0

        lax.fori_loop(0, KC // 2, step2, 0)

        @pl.when(c == 0)
        def _():
            for k in range(4):
                pltpu.sync_copy(e1_v, acc_sh.at[idx_v.at[k]], add=True)

        plsc.subcore_barrier()
        pltpu.sync_copy(acc_sh.at[pl.ds(s * RPT, RPT)], cnt_hbm.at[c, s])


def _make_sc_kernel(do_hist):
    mesh = plsc.VectorSubcoreMesh(core_axis_name="c", subcore_axis_name="s",
                                  num_cores=NC, num_subcores=NS)
    out_type = [
        jax.ShapeDtypeStruct((NC, NS, RPT, DIM), jnp.float32),
    ]
    scratch = [
        pltpu.VMEM_SHARED((N, DIM), jnp.float32),
        pltpu.VMEM((2, CB), jnp.int32),
        pltpu.VMEM((2, CB), jnp.int32),
        pltpu.VMEM((CB, DIM), jnp.float32),
        pltpu.VMEM((CB, DIM), jnp.float32),
    ]
    if do_hist:
        out_type.append(
            jax.ShapeDtypeStruct((NC, NS, RPT, DIM), jnp.float32))
        scratch += [
            pltpu.VMEM((CB, DIM), jnp.float32),
            pltpu.VMEM((16, DIM), jnp.float32),
            pltpu.VMEM((4, 16), jnp.int32),
        ]
    scratch += [pltpu.SemaphoreType.DMA] * 4
    return pl.kernel(functools.partial(_sc_body, do_hist), out_type=out_type,
                     mesh=mesh, scratch_types=scratch)


def _mm_body(x_ref, w_ref, o_ref):
    o_ref[...] = jnp.dot(x_ref[...], w_ref[...],
                         preferred_element_type=jnp.float32)


def _input_matmul(x, W):
    return pl.pallas_call(
        _mm_body,
        grid=(GRID,),
        in_specs=[pl.BlockSpec((BR, DIM), lambda i: (i, 0)),
                  pl.BlockSpec((DIM, DIM), lambda i: (0, 0))],
        out_specs=pl.BlockSpec((BR, DIM), lambda i: (i, 0)),
        out_shape=jax.ShapeDtypeStruct((N, DIM), jnp.float32),
    )(x, W)


def _film_body(p_ref, cnt_ref, hpre_ref, d_ref, g_ref, b_ref,
               w2_ref, bias_ref, h_ref, hnext_ref, lb_ref, lf_ref,
               u_acc, q_acc):
    i = pl.program_id(0)

    @pl.when(i == 0)
    def _():
        u_acc[...] = jnp.zeros_like(u_acc)
        q_acc[...] = jnp.zeros_like(q_acc)

    cnt = cnt_ref[0, :, 0:1] + cnt_ref[1, :, 0:1]
    sw = cnt_ref[0, :, 1:2]
    agg = (p_ref[0] + p_ref[1]) / jnp.maximum(cnt, 1.0)
    iota = lax.broadcasted_iota(jnp.int32, (BR, MAX_DEGREE), 1)
    onehot = (d_ref[...] == iota.astype(jnp.float32)).astype(jnp.float32)
    gamma = jnp.dot(onehot, g_ref[...], preferred_element_type=jnp.float32)
    beta = jnp.dot(onehot, b_ref[...], preferred_element_type=jnp.float32)
    out = gamma * agg + beta + hpre_ref[...]

    u_acc[...] += jnp.sum(sw * out, axis=0, keepdims=True)
    q_acc[...] += jnp.sum(sw * out * out, axis=0, keepdims=True)

    h = jnp.where(out > 0, out, 0.01 * out)
    h_ref[...] = h
    hnext_ref[...] = (jnp.dot(h, w2_ref[...],
                              preferred_element_type=jnp.float32)
                      + bias_ref[...])

    @pl.when(i == GRID - 1)
    def _():
        u = u_acc[...]
        q = q_acc[...]
        lb = (jnp.sum(q) - jnp.sum(u * u) / SAM) / (SAM * DIM)
        lf = ((jnp.sum(g_ref[...] * g_ref[...])
               + jnp.sum(b_ref[...] * b_ref[...]))
              / (MAX_DEGREE * DIM))
        lb_ref[...] = lb.reshape(1, 1)
        lf_ref[...] = lf.reshape(1, 1)


def _film_layer(parts, cnts, h_pre, d_f, gemb, bemb, W_next, bias_next):
    return pl.pallas_call(
        _film_body,
        grid=(GRID,),
        in_specs=[
            pl.BlockSpec((NC, BR, DIM), lambda i: (0, i, 0)),
            pl.BlockSpec((NC, BR, DIM), lambda i: (0, i, 0)),
            pl.BlockSpec((BR, DIM), lambda i: (i, 0)),
            pl.BlockSpec((BR, 1), lambda i: (i, 0)),
            pl.BlockSpec((MAX_DEGREE, DIM), lambda i: (0, 0)),
            pl.BlockSpec((MAX_DEGREE, DIM), lambda i: (0, 0)),
            pl.BlockSpec((DIM, DIM), lambda i: (0, 0)),
            pl.BlockSpec((1, DIM), lambda i: (0, 0)),
        ],
        out_specs=[
            pl.BlockSpec((BR, DIM), lambda i: (i, 0)),
            pl.BlockSpec((BR, DIM), lambda i: (i, 0)),
            pl.BlockSpec((1, 1), lambda i: (0, 0)),
            pl.BlockSpec((1, 1), lambda i: (0, 0)),
        ],
        out_shape=[
            jax.ShapeDtypeStruct((N, DIM), jnp.float32),
            jax.ShapeDtypeStruct((N, DIM), jnp.float32),
            jax.ShapeDtypeStruct((1, 1), jnp.float32),
            jax.ShapeDtypeStruct((1, 1), jnp.float32),
        ],
        scratch_shapes=[
            pltpu.VMEM((1, DIM), jnp.float32),
            pltpu.VMEM((1, DIM), jnp.float32),
        ],
    )(parts, cnts, h_pre, d_f, gemb, bemb, W_next, bias_next)


def kernel(x, edge_index, d, idx, W1, gamma1, beta1, W2, gamma2, beta2,
           fc_W, fc_b):
    src3 = edge_index[0].reshape(NW, KC, CB)
    dst3 = edge_index[1].reshape(NW, KC, CB)
    sd3 = jnp.stack((src3, dst3), axis=2)
    idx2 = idx.reshape(NS, 4, 16)
    d_f = d.astype(jnp.float32).reshape(N, 1)
    zeros = jnp.zeros((RPT, DIM), jnp.float32)
    ones0 = jnp.zeros((CB, DIM), jnp.float32).at[:, 0].set(1.0)
    e1 = jnp.zeros((16, DIM), jnp.float32).at[:, 1].set(1.0)
    fcWp = jnp.zeros((DIM, DIM), jnp.float32).at[:, :NCLASS].set(fc_W)
    fcbp = jnp.zeros((1, DIM), jnp.float32).at[0, :NCLASS].set(fc_b)
    zbias = jnp.zeros((1, DIM), jnp.float32)

    sc1 = _make_sc_kernel(True)
    sc2 = _make_sc_kernel(False)

    h1_pre = _input_matmul(x, W1)
    parts1, cnts = sc1(h1_pre, sd3, idx2, zeros, ones0, e1)
    parts1 = parts1.reshape(NC, N, DIM)
    cnts = cnts.reshape(NC, N, DIM)
    h1, h2_pre, b1, f1 = _film_layer(parts1, cnts, h1_pre, d_f,
                                     gamma1, beta1, W2, zbias)
    parts2 = sc2(h2_pre, sd3, idx2, zeros, ones0, e1)
    if isinstance(parts2, (list, tuple)):
        parts2 = parts2[0]
    parts2 = parts2.reshape(NC, N, DIM)
    _, logits_pad, b2, f2 = _film_layer(parts2, cnts, h2_pre, d_f,
                                        gamma2, beta2, fcWp, fcbp)

    logits = logits_pad[:, :NCLASS]
    return (logits, (b1 + b2)[0, 0], (f1 + f2)[0, 0], h1)
